# baseline (speedup 1.0000x reference)
import jax
import jax.numpy as jnp
from jax import lax
from jax.experimental import pallas as pl
from jax.experimental.pallas import tpu as pltpu
from jax.experimental.pallas import tpu_sc as plsc

N = 10000
D = 128
DOUT = 64
E = 320000

NC = 2
NS = 16
NW = NC * NS
CHUNK = 128
ECHUNKS = E // CHUNK
CHUNKS = 80
CPP = 20
NPAD = 10240
RPT = NPAD // NS
ZB = 40
DEGW = 16
BR = 2560

_sc_mesh = plsc.VectorSubcoreMesh(core_axis_name="c", subcore_axis_name="s")
_f32 = jnp.float32


def _fill(ref, rows, cols, val):
    @pl.loop(0, rows)
    def _(i):
        for j in range(cols // 16):
            ref[i, pl.ds(j * 16, 16)] = jnp.full((16,), val, _f32)


def _sc_agg_body(h_hbm, e_hbm, agg_hbm, ew, rows, zbuf, aggS,
                 gs0, gs1, ss0, ss1):
    cid = lax.axis_index("c")
    sid = lax.axis_index("s")
    wid = cid * NS + sid
    gsems = (gs0, gs1)
    ssems = (ss0, ss1)

    _fill(zbuf, ZB, D, 0.0)

    @pl.loop(0, RPT // ZB)
    def _(k):
        pltpu.sync_copy(zbuf, aggS.at[pl.ds(sid * RPT + k * ZB, ZB)])

    plsc.subcore_barrier()

    nph = jnp.where(wid < NW - 1, CHUNKS // CPP, 1)

    @pl.loop(0, nph)
    def _(p):
        pltpu.sync_copy(e_hbm.at[pl.ds(wid * CHUNKS + p * CPP, CPP)], ew)
        pltpu.async_copy(h_hbm.at[ew.at[0, 0]], rows.at[0], gs0)
        pltpu.async_copy(h_hbm.at[ew.at[1, 0]], rows.at[1], gs1)

        @pl.loop(0, CPP, step=2)
        def _(j):
            for b in range(2):
                jb = j + b
                pltpu.make_async_copy(
                    h_hbm.at[ew.at[jb, 0]], rows.at[b], gsems[b]).wait()
                pltpu.async_copy(rows.at[b], aggS.at[ew.at[jb, 1]],
                                 ssems[b], add=True)
            for b in range(2):
                jb = j + b
                pltpu.make_async_copy(rows.at[b], aggS.at[ew.at[jb, 1]],
                                      ssems[b]).wait()

                @pl.when(jb + 2 < CPP)
                def _():
                    pltpu.async_copy(
                        h_hbm.at[ew.at[jb + 2, 0]], rows.at[b], gsems[b])

    plsc.subcore_barrier()

    pltpu.sync_copy(aggS.at[pl.ds(sid * RPT, RPT)],
                    agg_hbm.at[cid].at[pl.ds(sid * RPT, RPT)])


_agg = pl.kernel(
    _sc_agg_body,
    out_type=jax.ShapeDtypeStruct((NC, NPAD, D), _f32),
    mesh=_sc_mesh,
    scratch_types=[
        pltpu.VMEM((CPP, 2, CHUNK), jnp.int32),
        pltpu.VMEM((2, CHUNK, D), _f32),
        pltpu.VMEM((ZB, D), _f32),
        pltpu.VMEM_SHARED((NPAD, D), _f32),
        pltpu.SemaphoreType.DMA,
        pltpu.SemaphoreType.DMA,
        pltpu.SemaphoreType.DMA,
        pltpu.SemaphoreType.DMA,
    ],
)


def _sc_deg_body(e_hbm, deg_hbm, ew, onev, zbuf, degS):
    cid = lax.axis_index("c")
    sid = lax.axis_index("s")
    wid = cid * NS + sid

    _fill(onev, CHUNK, D, 1.0)
    _fill(zbuf, ZB, D, 0.0)

    @pl.loop(0, RPT // ZB)
    def _(k):
        pltpu.sync_copy(zbuf, degS.at[pl.ds(sid * RPT + k * ZB, ZB)])

    plsc.subcore_barrier()

    nph = jnp.where(wid < NW - 1, CHUNKS // CPP, 1)

    @pl.loop(0, nph)
    def _(p):
        pltpu.sync_copy(e_hbm.at[pl.ds(wid * CHUNKS + p * CPP, CPP)], ew)

        @pl.loop(0, CPP)
        def _(j):
            pltpu.sync_copy(onev, degS.at[ew.at[j, 1]], add=True)

    plsc.subcore_barrier()
    pltpu.sync_copy(degS.at[pl.ds(sid * RPT, RPT)],
                    deg_hbm.at[cid].at[pl.ds(sid * RPT, RPT)])


_deg = pl.kernel(
    _sc_deg_body,
    out_type=jax.ShapeDtypeStruct((NC, NPAD, D), _f32),
    mesh=_sc_mesh,
    scratch_types=[
        pltpu.VMEM((CPP, 2, CHUNK), jnp.int32),
        pltpu.VMEM((CHUNK, D), _f32),
        pltpu.VMEM((ZB, D), _f32),
        pltpu.VMEM_SHARED((NPAD, D), _f32),
    ],
)


def _l2norm(x):
    def body(x_ref, o_ref):
        xv = x_ref[...]
        nrm = jnp.sqrt(jnp.sum(xv * xv, axis=1, keepdims=True))
        o_ref[...] = xv / jnp.maximum(nrm, 1e-12)

    return pl.pallas_call(
        body,
        out_shape=jax.ShapeDtypeStruct((NPAD, D), _f32),
        grid=(NPAD // BR,),
        in_specs=[pl.BlockSpec((BR, D), lambda i: (i, 0))],
        out_specs=pl.BlockSpec((BR, D), lambda i: (i, 0)),
    )(x)


def _sage_mean_layer(h, aggP, inv_b, wsT, wnT, b, dout, relu):
    def body(h_ref, agg_ref, inv_ref, ws_ref, wn_ref, b_ref, o_ref):
        hv = h_ref[...]
        hn = (agg_ref[0] + agg_ref[1]) * inv_ref[...]
        acc = jnp.dot(hv, ws_ref[...], preferred_element_type=_f32)
        acc = acc + jnp.dot(hn, wn_ref[...], preferred_element_type=_f32)
        acc = acc + b_ref[...]
        if relu:
            acc = jnp.maximum(acc, 0.0)
        o_ref[...] = acc

    return pl.pallas_call(
        body,
        out_shape=jax.ShapeDtypeStruct((NPAD, dout), _f32),
        grid=(NPAD // BR,),
        in_specs=[
            pl.BlockSpec((BR, D), lambda i: (i, 0)),
            pl.BlockSpec((NC, BR, D), lambda i: (0, i, 0)),
            pl.BlockSpec((BR, D), lambda i: (i, 0)),
            pl.BlockSpec((D, dout), lambda i: (0, 0)),
            pl.BlockSpec((D, dout), lambda i: (0, 0)),
            pl.BlockSpec((1, dout), lambda i: (0, 0)),
        ],
        out_specs=pl.BlockSpec((BR, dout), lambda i: (i, 0)),
    )(h, aggP, inv_b, wsT, wnT, b)


def _sage_gcn_layer(h, aggP, invh_b, wT, b):
    def body(h_ref, agg_ref, inv_ref, w_ref, b_ref, o_ref):
        rst = (h_ref[...] + agg_ref[0] + agg_ref[1]) * inv_ref[...]
        acc = jnp.dot(rst, w_ref[...], preferred_element_type=_f32)
        o_ref[...] = jnp.maximum(acc + b_ref[...], 0.0)

    return pl.pallas_call(
        body,
        out_shape=jax.ShapeDtypeStruct((NPAD, D), _f32),
        grid=(NPAD // BR,),
        in_specs=[
            pl.BlockSpec((BR, D), lambda i: (i, 0)),
            pl.BlockSpec((NC, BR, D), lambda i: (0, i, 0)),
            pl.BlockSpec((BR, D), lambda i: (i, 0)),
            pl.BlockSpec((D, D), lambda i: (0, 0)),
            pl.BlockSpec((1, D), lambda i: (0, 0)),
        ],
        out_specs=pl.BlockSpec((BR, D), lambda i: (i, 0)),
    )(h, aggP, invh_b, wT, b)


def kernel(x, edge_index, Ws0, Wn0, b0, W1, b1, W2, b2, Ws3, Wn3, b3):
    e3 = jnp.transpose(
        edge_index.astype(jnp.int32).reshape(2, ECHUNKS, CHUNK), (1, 0, 2))

    x_pad = jnp.zeros((NPAD, D), _f32).at[:N].set(x.astype(_f32))

    degP = _deg(e3)
    h0 = _l2norm(x_pad)
    deg = degP[0, :, 0] + degP[1, :, 0]
    inv0 = jnp.broadcast_to((1.0 / jnp.maximum(deg, 1.0))[:, None], (NPAD, D))
    invh = jnp.broadcast_to((1.0 / (deg + 1.0))[:, None], (NPAD, D))

    aggP0 = _agg(h0, e3)
    h1 = _sage_mean_layer(h0, aggP0, inv0, Ws0.T, Wn0.T,
                          b0.reshape(1, D), D, True)
    aggP1 = _agg(h1, e3)
    h2 = _sage_gcn_layer(h1, aggP1, invh, W1.T, b1.reshape(1, D))
    aggP2 = _agg(h2, e3)
    h3 = _sage_gcn_layer(h2, aggP2, invh, W2.T, b2.reshape(1, D))
    aggP3 = _agg(h3, e3)
    out = _sage_mean_layer(h3, aggP3, inv0, Ws3.T, Wn3.T,
                           b3.reshape(1, DOUT), DOUT, False)
    return out[:N]


# speedup vs baseline: 1.2255x; 1.2255x over previous
import jax
import jax.numpy as jnp
from jax import lax
from jax.experimental import pallas as pl
from jax.experimental.pallas import tpu as pltpu
from jax.experimental.pallas import tpu_sc as plsc

N = 10000
D = 128
DOUT = 64
E = 320000

NC = 2
NS = 16
NW = NC * NS
CHUNK = 128
ECHUNKS = E // CHUNK
CHUNKS = 80
CPP = 20
NPAD = 10240
RPT = NPAD // NS
ZB = 40
DEGW = 16
BR = 2560

_sc_mesh = plsc.VectorSubcoreMesh(core_axis_name="c", subcore_axis_name="s")
_f32 = jnp.float32


def _fill(ref, rows, cols, val):
    @pl.loop(0, rows)
    def _(i):
        for j in range(cols // 16):
            ref[i, pl.ds(j * 16, 16)] = jnp.full((16,), val, _f32)


def _sc_agg_body(h_hbm, e_hbm, agg_hbm, ew, rows, zbuf, aggS, gs0, gs1):
    cid = lax.axis_index("c")
    sid = lax.axis_index("s")
    wid = cid * NS + sid
    gsems = (gs0, gs1)

    _fill(zbuf, ZB, D, 0.0)

    @pl.loop(0, RPT // ZB)
    def _(k):
        pltpu.sync_copy(zbuf, aggS.at[pl.ds(sid * RPT + k * ZB, ZB)])

    plsc.subcore_barrier()

    nph = jnp.where(wid < NW - 1, CHUNKS // CPP, 1)

    @pl.loop(0, nph)
    def _(p):
        pltpu.sync_copy(e_hbm.at[pl.ds(wid * CHUNKS + p * CPP, CPP)], ew)
        pltpu.async_copy(h_hbm.at[ew.at[0, 0]], rows.at[0], gs0)
        pltpu.async_copy(h_hbm.at[ew.at[1, 0]], rows.at[1], gs1)

        @pl.loop(0, CPP, step=2)
        def _(j):
            for b in range(2):
                jb = j + b
                pltpu.make_async_copy(
                    h_hbm.at[ew.at[jb, 0]], rows.at[b], gsems[b]).wait()
                pltpu.sync_copy(rows.at[b], aggS.at[ew.at[jb, 1]], add=True)

                @pl.when(jb + 2 < CPP)
                def _():
                    pltpu.async_copy(
                        h_hbm.at[ew.at[jb + 2, 0]], rows.at[b], gsems[b])

    plsc.subcore_barrier()

    pltpu.sync_copy(aggS.at[pl.ds(sid * RPT, RPT)],
                    agg_hbm.at[cid].at[pl.ds(sid * RPT, RPT)])


_agg = pl.kernel(
    _sc_agg_body,
    out_type=jax.ShapeDtypeStruct((NC, NPAD, D), _f32),
    mesh=_sc_mesh,
    scratch_types=[
        pltpu.VMEM((CPP, 2, CHUNK), jnp.int32),
        pltpu.VMEM((2, CHUNK, D), _f32),
        pltpu.VMEM((ZB, D), _f32),
        pltpu.VMEM_SHARED((NPAD, D), _f32),
        pltpu.SemaphoreType.DMA,
        pltpu.SemaphoreType.DMA,
    ],
)


def _sc_deg_body(e_hbm, deg_hbm, ew, onev, zbuf, degS):
    cid = lax.axis_index("c")
    sid = lax.axis_index("s")
    wid = cid * NS + sid

    _fill(onev, CHUNK, D, 1.0)
    _fill(zbuf, ZB, D, 0.0)

    @pl.loop(0, RPT // ZB)
    def _(k):
        pltpu.sync_copy(zbuf, degS.at[pl.ds(sid * RPT + k * ZB, ZB)])

    plsc.subcore_barrier()

    nph = jnp.where(wid < NW - 1, CHUNKS // CPP, 1)

    @pl.loop(0, nph)
    def _(p):
        pltpu.sync_copy(e_hbm.at[pl.ds(wid * CHUNKS + p * CPP, CPP)], ew)

        @pl.loop(0, CPP)
        def _(j):
            pltpu.sync_copy(onev, degS.at[ew.at[j, 1]], add=True)

    plsc.subcore_barrier()
    pltpu.sync_copy(degS.at[pl.ds(sid * RPT, RPT)],
                    deg_hbm.at[cid].at[pl.ds(sid * RPT, RPT)])


_deg = pl.kernel(
    _sc_deg_body,
    out_type=jax.ShapeDtypeStruct((NC, NPAD, D), _f32),
    mesh=_sc_mesh,
    scratch_types=[
        pltpu.VMEM((CPP, 2, CHUNK), jnp.int32),
        pltpu.VMEM((CHUNK, D), _f32),
        pltpu.VMEM((ZB, D), _f32),
        pltpu.VMEM_SHARED((NPAD, D), _f32),
    ],
)


def _l2norm(x):
    def body(x_ref, o_ref):
        xv = x_ref[...]
        nrm = jnp.sqrt(jnp.sum(xv * xv, axis=1, keepdims=True))
        o_ref[...] = xv / jnp.maximum(nrm, 1e-12)

    return pl.pallas_call(
        body,
        out_shape=jax.ShapeDtypeStruct((NPAD, D), _f32),
        grid=(NPAD // BR,),
        in_specs=[pl.BlockSpec((BR, D), lambda i: (i, 0))],
        out_specs=pl.BlockSpec((BR, D), lambda i: (i, 0)),
    )(x)


def _sage_mean_layer(h, aggP, inv_b, wsT, wnT, b, dout, relu):
    def body(h_ref, agg_ref, inv_ref, ws_ref, wn_ref, b_ref, o_ref):
        hv = h_ref[...]
        hn = (agg_ref[0] + agg_ref[1]) * inv_ref[...]
        acc = jnp.dot(hv, ws_ref[...], preferred_element_type=_f32)
        acc = acc + jnp.dot(hn, wn_ref[...], preferred_element_type=_f32)
        acc = acc + b_ref[...]
        if relu:
            acc = jnp.maximum(acc, 0.0)
        o_ref[...] = acc

    return pl.pallas_call(
        body,
        out_shape=jax.ShapeDtypeStruct((NPAD, dout), _f32),
        grid=(NPAD // BR,),
        in_specs=[
            pl.BlockSpec((BR, D), lambda i: (i, 0)),
            pl.BlockSpec((NC, BR, D), lambda i: (0, i, 0)),
            pl.BlockSpec((BR, D), lambda i: (i, 0)),
            pl.BlockSpec((D, dout), lambda i: (0, 0)),
            pl.BlockSpec((D, dout), lambda i: (0, 0)),
            pl.BlockSpec((1, dout), lambda i: (0, 0)),
        ],
        out_specs=pl.BlockSpec((BR, dout), lambda i: (i, 0)),
    )(h, aggP, inv_b, wsT, wnT, b)


def _sage_gcn_layer(h, aggP, invh_b, wT, b):
    def body(h_ref, agg_ref, inv_ref, w_ref, b_ref, o_ref):
        rst = (h_ref[...] + agg_ref[0] + agg_ref[1]) * inv_ref[...]
        acc = jnp.dot(rst, w_ref[...], preferred_element_type=_f32)
        o_ref[...] = jnp.maximum(acc + b_ref[...], 0.0)

    return pl.pallas_call(
        body,
        out_shape=jax.ShapeDtypeStruct((NPAD, D), _f32),
        grid=(NPAD // BR,),
        in_specs=[
            pl.BlockSpec((BR, D), lambda i: (i, 0)),
            pl.BlockSpec((NC, BR, D), lambda i: (0, i, 0)),
            pl.BlockSpec((BR, D), lambda i: (i, 0)),
            pl.BlockSpec((D, D), lambda i: (0, 0)),
            pl.BlockSpec((1, D), lambda i: (0, 0)),
        ],
        out_specs=pl.BlockSpec((BR, D), lambda i: (i, 0)),
    )(h, aggP, invh_b, wT, b)


def kernel(x, edge_index, Ws0, Wn0, b0, W1, b1, W2, b2, Ws3, Wn3, b3):
    e3 = jnp.transpose(
        edge_index.astype(jnp.int32).reshape(2, ECHUNKS, CHUNK), (1, 0, 2))

    x_pad = jnp.zeros((NPAD, D), _f32).at[:N].set(x.astype(_f32))

    degP = _deg(e3)
    h0 = _l2norm(x_pad)
    deg = degP[0, :, 0] + degP[1, :, 0]
    inv0 = jnp.broadcast_to((1.0 / jnp.maximum(deg, 1.0))[:, None], (NPAD, D))
    invh = jnp.broadcast_to((1.0 / (deg + 1.0))[:, None], (NPAD, D))

    aggP0 = _agg(h0, e3)
    h1 = _sage_mean_layer(h0, aggP0, inv0, Ws0.T, Wn0.T,
                          b0.reshape(1, D), D, True)
    aggP1 = _agg(h1, e3)
    h2 = _sage_gcn_layer(h1, aggP1, invh, W1.T, b1.reshape(1, D))
    aggP2 = _agg(h2, e3)
    h3 = _sage_gcn_layer(h2, aggP2, invh, W2.T, b2.reshape(1, D))
    aggP3 = _agg(h3, e3)
    out = _sage_mean_layer(h3, aggP3, inv0, Ws3.T, Wn3.T,
                           b3.reshape(1, DOUT), DOUT, False)
    return out[:N]
